# baseline (speedup 1.0000x reference)
import functools

import jax
import jax.numpy as jnp
from jax import lax
from jax.experimental import pallas as pl
from jax.experimental.pallas import tpu as pltpu
from jax.experimental.pallas import tpu_sc as plsc

_K = 16
_CODE = 128
_LANES = 16
_CHUNKS = _CODE // _LANES
_NC, _NS = 2, 16
_NW = _NC * _NS


def _mm_body(x_ref, z_ref, w_ref, s_ref, o_ref):
    u = jax.lax.dot_general(
        x_ref[...], w_ref[...], (((1,), (1,)), ((), ())),
        preferred_element_type=jnp.float32)
    o_ref[...] = u + jax.lax.dot_general(
        z_ref[...], s_ref[...], (((1,), (1,)), ((), ())),
        preferred_element_type=jnp.float32)


def _matmul(x, z_prev, W, S):
    n, d_in = x.shape
    code = W.shape[0]
    blk = 1024
    return pl.pallas_call(
        _mm_body,
        grid=(n // blk,),
        in_specs=[
            pl.BlockSpec((blk, d_in), lambda i: (i, 0)),
            pl.BlockSpec((blk, code), lambda i: (i, 0)),
            pl.BlockSpec((code, d_in), lambda i: (0, 0)),
            pl.BlockSpec((code, code), lambda i: (0, 0)),
        ],
        out_specs=pl.BlockSpec((blk, code), lambda i: (i, 0)),
        out_shape=jax.ShapeDtypeStruct((n, code), jnp.float32),
    )(x, z_prev, W, S)


def _row_threshold(a):
    s0 = [plsc.sort_key_val(c, c, descending=(j % 2 == 0))[0]
          for j, c in enumerate(a)]
    m1 = [jnp.maximum(s0[2 * i], s0[2 * i + 1]) for i in range(4)]
    s1 = [plsc.sort_key_val(c, c, descending=(i % 2 == 0))[0]
          for i, c in enumerate(m1)]
    m2 = [jnp.maximum(s1[2 * i], s1[2 * i + 1]) for i in range(2)]
    s2 = [plsc.sort_key_val(c, c, descending=(i % 2 == 0))[0]
          for i, c in enumerate(m2)]
    m3 = jnp.maximum(s2[0], s2[1])
    return jnp.min(m3)


def _sc_mask_body(u_hbm, out_hbm, buf, rows_per_w):
    wid = lax.axis_index("s") * _NC + lax.axis_index("c")
    base = wid * rows_per_w
    pltpu.sync_copy(u_hbm.at[pl.ds(base, rows_per_w)], buf)

    @plsc.parallel_loop(0, rows_per_w, unroll=8)
    def row_body(r):
        v = [buf[r, pl.ds(j * _LANES, _LANES)] for j in range(_CHUNKS)]
        a = [jnp.abs(c) for c in v]
        t = _row_threshold(a)
        for j in range(_CHUNKS):
            buf[r, pl.ds(j * _LANES, _LANES)] = jnp.where(
                a[j] >= t, v[j], 0.0)
    pltpu.sync_copy(buf, out_hbm.at[pl.ds(base, rows_per_w)])


def _sc_mask(update):
    n, code = update.shape
    rows_per_w = n // _NW
    mesh = plsc.VectorSubcoreMesh(
        core_axis_name="c", subcore_axis_name="s",
        num_cores=_NC, num_subcores=_NS)
    body = functools.partial(_sc_mask_body, rows_per_w=rows_per_w)
    f = pl.kernel(
        body,
        out_type=jax.ShapeDtypeStruct((n, code), jnp.float32),
        mesh=mesh,
        scratch_types=[pltpu.VMEM((rows_per_w, code), jnp.float32)],
        compiler_params=pltpu.CompilerParams(needs_layout_passes=False),
    )
    return f(update)


@jax.jit
def kernel(x, z_prev, W, S):
    update = _matmul(x, z_prev, W, S)
    return _sc_mask(update)


# speedup vs baseline: 2.3247x; 2.3247x over previous
import jax
import jax.numpy as jnp
from jax.experimental import pallas as pl

_K = 16


def _body(x_ref, z_ref, w_ref, s_ref, o_ref):
    x = x_ref[...]
    z = z_ref[...]
    w = w_ref[...]
    s = s_ref[...]
    ut = jax.lax.dot_general(w, x, (((1,), (1,)), ((), ())),
                             preferred_element_type=jnp.float32)
    ut = ut + jax.lax.dot_general(s, z, (((1,), (1,)), ((), ())),
                                  preferred_element_type=jnp.float32)
    b = jnp.abs(ut)
    m = None
    for i in range(_K):
        m = jnp.max(b, axis=0, keepdims=True)
        if i < _K - 1:
            b = jnp.where(b >= m, -1.0, b)
    u = jax.lax.dot_general(x, w, (((1,), (1,)), ((), ())),
                            preferred_element_type=jnp.float32)
    u = u + jax.lax.dot_general(z, s, (((1,), (1,)), ((), ())),
                                preferred_element_type=jnp.float32)
    tcol = jax.lax.transpose(m, (1, 0))
    o_ref[...] = jnp.where(jnp.abs(u) >= tcol, u, 0.0)


@jax.jit
def kernel(x, z_prev, W, S):
    n, d_in = x.shape
    code = W.shape[0]
    blk = 4096
    return pl.pallas_call(
        _body,
        grid=(n // blk,),
        in_specs=[
            pl.BlockSpec((blk, d_in), lambda i: (i, 0)),
            pl.BlockSpec((blk, code), lambda i: (i, 0)),
            pl.BlockSpec((code, d_in), lambda i: (0, 0)),
            pl.BlockSpec((code, code), lambda i: (0, 0)),
        ],
        out_specs=pl.BlockSpec((blk, code), lambda i: (i, 0)),
        out_shape=jax.ShapeDtypeStruct((n, code), jnp.float32),
    )(x, z_prev, W, S)
